# baseline (speedup 1.0000x reference)
import functools

import jax
import jax.numpy as jnp
from jax import lax
from jax.experimental import pallas as pl
from jax.experimental.pallas import tpu as pltpu
from jax.experimental.pallas import tpu_sc as plsc

B_TOK = 8192
D = 256
K = 8192
COMMIT_W = 0.25

BX = 2048
BC = 4096
NXB = B_TOK // BX
NCB = K // BC

SC_CORES = 2
SC_SUBCORES = 16
NW = SC_CORES * SC_SUBCORES
BPW = B_TOK // NW
GCHUNK = 128


def _dist_argmin_body(x_ref, cb_ref, prev_ref, ids_ref, loss_ref, x2,
                      minval, minidx, acc, xsum, *, nxb, final):
    i = pl.program_id(0)
    j = pl.program_id(1)

    @pl.when(j == 0)
    def _():
        xx = x_ref[...]
        x2[...] = xx * -2.0
        xsum[0, 0] = jnp.sum(xx * xx)

    c = cb_ref[...]
    cn = jnp.sum(c * c, axis=1, keepdims=True)
    xc2 = lax.dot_general(
        c, x2[...], (((1,), (1,)), ((), ())),
        preferred_element_type=jnp.float32,
        precision=lax.Precision.DEFAULT,
    )
    distT = xc2 + cn
    bmin = jnp.min(distT, axis=0)
    bidx = jnp.argmin(distT, axis=0).astype(jnp.int32) + j * BC

    @pl.when(j == 0)
    def _():
        minval[...] = bmin
        minidx[...] = bidx

    @pl.when(j > 0)
    def _():
        better = bmin < minval[...]
        minval[...] = jnp.where(better, bmin, minval[...])
        minidx[...] = jnp.where(better, bidx, minidx[...])

    @pl.when(j == NCB - 1)
    def _():
        ids_ref[...] = minidx[...]
        part = jnp.sum(minval[...]) + xsum[0, 0]
        prev = prev_ref[0, 0] if prev_ref is not None else 0.0
        tot = jnp.where(i == 0, prev + part, acc[0, 0] + part)
        acc[0, 0] = tot

        @pl.when(i == nxb - 1)
        def _():
            if final:
                loss_ref[0, 0] = tot * ((1.0 + COMMIT_W) / B_TOK)
            else:
                loss_ref[0, 0] = tot


def _dist_argmin(x, codebook, prev=None, *, final):
    ntok = x.shape[0]
    nxb = ntok // BX
    body = functools.partial(_dist_argmin_body, nxb=nxb, final=final)
    if prev is None:
        body = lambda x_ref, cb_ref, *rest: functools.partial(
            _dist_argmin_body, nxb=nxb, final=final)(x_ref, cb_ref, None, *rest)
        in_specs = [
            pl.BlockSpec((BX, D), lambda i, j: (i, 0)),
            pl.BlockSpec((BC, D), lambda i, j: (j, 0)),
        ]
        args = (x, codebook)
    else:
        in_specs = [
            pl.BlockSpec((BX, D), lambda i, j: (i, 0)),
            pl.BlockSpec((BC, D), lambda i, j: (j, 0)),
            pl.BlockSpec(memory_space=pltpu.SMEM),
        ]
        args = (x, codebook, prev)
    return pl.pallas_call(
        body,
        grid=(nxb, NCB),
        in_specs=in_specs,
        out_specs=[
            pl.BlockSpec((BX,), lambda i, j: (i,)),
            pl.BlockSpec(memory_space=pltpu.SMEM),
        ],
        out_shape=[
            jax.ShapeDtypeStruct((ntok,), jnp.int32),
            jax.ShapeDtypeStruct((1, 1), jnp.float32),
        ],
        scratch_shapes=[
            pltpu.VMEM((BX, D), jnp.float32),
            pltpu.VMEM((BX,), jnp.float32),
            pltpu.VMEM((BX,), jnp.int32),
            pltpu.SMEM((1, 1), jnp.float32),
            pltpu.SMEM((1, 1), jnp.float32),
        ],
    )(*args)


def _gather_body(cb_hbm, ids_hbm, out_hbm, idx_v, rows_v, sem, *, bpw):
    wid = lax.axis_index("s") * SC_CORES + lax.axis_index("c")
    base = wid * bpw
    for t in range(bpw // GCHUNK):
        off = base + t * GCHUNK
        pltpu.sync_copy(ids_hbm.at[pl.ds(off, GCHUNK)], idx_v)
        pltpu.async_copy(cb_hbm.at[idx_v], rows_v, sem).wait()
        pltpu.sync_copy(rows_v, out_hbm.at[pl.ds(off, GCHUNK)])


def _sc_gather(codebook, ids):
    ntok = ids.shape[0]
    bpw = ntok // NW
    mesh = plsc.VectorSubcoreMesh(
        core_axis_name="c", subcore_axis_name="s",
        num_cores=SC_CORES, num_subcores=SC_SUBCORES,
    )
    return pl.kernel(
        functools.partial(_gather_body, bpw=bpw),
        out_type=jax.ShapeDtypeStruct((ntok, D), jnp.float32),
        mesh=mesh,
        scratch_types=[
            pltpu.VMEM((GCHUNK,), jnp.int32),
            pltpu.VMEM((GCHUNK, D), jnp.float32),
            pltpu.SemaphoreType.DMA,
        ],
    )(codebook, ids)


def kernel(x, temperature, codebook):
    half = B_TOK // 2
    ids_a, sum_a = _dist_argmin(x[:half], codebook, final=False)
    ids_b, loss = _dist_argmin(x[half:], codebook, sum_a, final=True)
    emb_a = _sc_gather(codebook, ids_a)
    emb_b = _sc_gather(codebook, ids_b)
    ids = jnp.concatenate([ids_a, ids_b])
    emb_out = jnp.concatenate([emb_a, emb_b], axis=0)
    return emb_out, ids, loss.reshape(())


# speedup vs baseline: 1.1395x; 1.1395x over previous
import functools

import jax
import jax.numpy as jnp
from jax import lax
from jax.experimental import pallas as pl
from jax.experimental.pallas import tpu as pltpu
from jax.experimental.pallas import tpu_sc as plsc

B_TOK = 8192
D = 256
K = 8192
COMMIT_W = 0.25

BX = 1024
BC = 4096
NXB = B_TOK // BX
NCB = K // BC

SC_CORES = 2
SC_SUBCORES = 16
NW = SC_CORES * SC_SUBCORES
BPW = B_TOK // NW
GCHUNK = 128


def _dist_argmin_body(x_ref, cb_ref, ids_ref, x2, cn_sc):
    i = pl.program_id(0)

    xx = x_ref[...]
    x2[...] = xx * -2.0

    c = cb_ref[...]

    @pl.when(i == 0)
    def _():
        cn_sc[...] = jnp.sum(c * c, axis=1, keepdims=True)

    xc2 = lax.dot_general(
        c, x2[...], (((1,), (1,)), ((), ())),
        preferred_element_type=jnp.float32,
        precision=lax.Precision.DEFAULT,
    )
    distT = xc2 + cn_sc[...]
    ids_ref[...] = jnp.argmin(distT, axis=0).astype(jnp.int32)


def _dist_argmin(x, codebook):
    nxb = B_TOK // BX
    return pl.pallas_call(
        _dist_argmin_body,
        grid=(nxb,),
        in_specs=[
            pl.BlockSpec((BX, D), lambda i: (i, 0)),
            pl.BlockSpec((K, D), lambda i: (0, 0)),
        ],
        out_specs=pl.BlockSpec((BX,), lambda i: (i,)),
        out_shape=jax.ShapeDtypeStruct((B_TOK,), jnp.int32),
        scratch_shapes=[
            pltpu.VMEM((BX, D), jnp.float32),
            pltpu.VMEM((K, 1), jnp.float32),
        ],
    )(x, codebook)


def _gather_loss_body(cb_hbm, x_hbm, ids_hbm, out_hbm, part_hbm,
                      idx0, idx1, rows0, rows1, xv, accv, tmpv, shared,
                      semg0, semg1, semx, semo0, semo1):
    cid = lax.axis_index("c")
    sid = lax.axis_index("s")
    wid = sid * SC_CORES + cid
    base = wid * BPW

    pltpu.sync_copy(ids_hbm.at[pl.ds(base, GCHUNK)], idx0)
    g0 = pltpu.async_copy(cb_hbm.at[idx0], rows0, semg0)
    pltpu.sync_copy(ids_hbm.at[pl.ds(base + GCHUNK, GCHUNK)], idx1)
    g1 = pltpu.async_copy(cb_hbm.at[idx1], rows1, semg1)

    def sq_acc(rows_ref, acc):
        def body(r, acc):
            a = acc
            for v in range(D // 16):
                dlt = xv[r, pl.ds(16 * v, 16)] - rows_ref[r, pl.ds(16 * v, 16)]
                a = a + dlt * dlt
            return a
        return lax.fori_loop(0, GCHUNK, body, acc)

    x0 = pltpu.async_copy(x_hbm.at[pl.ds(base, GCHUNK)], xv, semx)
    g0.wait()
    o0 = pltpu.async_copy(rows0, out_hbm.at[pl.ds(base, GCHUNK)], semo0)
    x0.wait()
    acc = sq_acc(rows0, jnp.zeros((16,), jnp.float32))
    g1.wait()
    o1 = pltpu.async_copy(rows1, out_hbm.at[pl.ds(base + GCHUNK, GCHUNK)],
                          semo1)
    pltpu.sync_copy(x_hbm.at[pl.ds(base + GCHUNK, GCHUNK)], xv)
    acc = sq_acc(rows1, acc)
    accv[...] = acc
    o0.wait()
    o1.wait()

    pltpu.sync_copy(accv, shared.at[sid])
    plsc.subcore_barrier()

    @pl.when(sid == 0)
    def _():
        tot = jnp.zeros((16,), jnp.float32)
        for sc in range(SC_SUBCORES):
            pltpu.sync_copy(shared.at[sc], tmpv)
            tot = tot + tmpv[...]
        tmpv[...] = tot * ((1.0 + COMMIT_W) / B_TOK)
        pltpu.sync_copy(tmpv, part_hbm.at[cid])


def _sc_gather_loss(codebook, x, ids):
    mesh = plsc.VectorSubcoreMesh(
        core_axis_name="c", subcore_axis_name="s",
        num_cores=SC_CORES, num_subcores=SC_SUBCORES,
    )
    return pl.kernel(
        _gather_loss_body,
        out_type=(
            jax.ShapeDtypeStruct((B_TOK, D), jnp.float32),
            jax.ShapeDtypeStruct((SC_CORES, 16), jnp.float32),
        ),
        mesh=mesh,
        scratch_types=[
            pltpu.VMEM((GCHUNK,), jnp.int32),
            pltpu.VMEM((GCHUNK,), jnp.int32),
            pltpu.VMEM((GCHUNK, D), jnp.float32),
            pltpu.VMEM((GCHUNK, D), jnp.float32),
            pltpu.VMEM((GCHUNK, D), jnp.float32),
            pltpu.VMEM((16,), jnp.float32),
            pltpu.VMEM((16,), jnp.float32),
            pltpu.VMEM_SHARED((SC_SUBCORES, 16), jnp.float32),
            pltpu.SemaphoreType.DMA,
            pltpu.SemaphoreType.DMA,
            pltpu.SemaphoreType.DMA,
            pltpu.SemaphoreType.DMA,
            pltpu.SemaphoreType.DMA,
        ],
    )(codebook, x, ids)


def kernel(x, temperature, codebook):
    ids = _dist_argmin(x, codebook)
    emb_out, part = _sc_gather_loss(codebook, x, ids)
    loss = jnp.sum(part)
    return emb_out, ids, loss


# speedup vs baseline: 1.1754x; 1.0315x over previous
import functools

import jax
import jax.numpy as jnp
from jax import lax
from jax.experimental import pallas as pl
from jax.experimental.pallas import tpu as pltpu
from jax.experimental.pallas import tpu_sc as plsc

B_TOK = 8192
D = 256
K = 8192
COMMIT_W = 0.25

BX = 1024
BC = 4096
NXB = B_TOK // BX
NCB = K // BC

SC_CORES = 2
SC_SUBCORES = 16
NW = SC_CORES * SC_SUBCORES
BPW = B_TOK // NW
GCHUNK = 128


def _dist_argmin_body(x_ref, cb_ref, ids_ref, x2, cn_sc):
    i = pl.program_id(0)

    xx = x_ref[...]
    x2[...] = xx * -2.0

    c = cb_ref[...]

    @pl.when(i == 0)
    def _():
        cn_sc[...] = jnp.sum(c * c, axis=1, keepdims=True)

    xc2 = lax.dot_general(
        c, x2[...], (((1,), (1,)), ((), ())),
        preferred_element_type=jnp.float32,
        precision=lax.Precision.DEFAULT,
    )
    distT = xc2 + cn_sc[...]
    ids_ref[...] = jnp.argmin(distT, axis=0).astype(jnp.int32)


def _dist_argmin(x, codebook):
    nxb = B_TOK // BX
    return pl.pallas_call(
        _dist_argmin_body,
        grid=(nxb,),
        in_specs=[
            pl.BlockSpec((BX, D), lambda i: (i, 0)),
            pl.BlockSpec((K, D), lambda i: (0, 0)),
        ],
        out_specs=pl.BlockSpec((BX,), lambda i: (i,)),
        out_shape=jax.ShapeDtypeStruct((B_TOK,), jnp.int32),
        scratch_shapes=[
            pltpu.VMEM((BX, D), jnp.float32),
            pltpu.VMEM((K, 1), jnp.float32),
        ],
    )(x, codebook)


def _gather_loss_body(cb_hbm, x_hbm, ids_hbm, out_hbm, part_hbm,
                      idx0, idx1, rows, xv, accv, tmpv, shared,
                      semg, semx, semo):
    cid = lax.axis_index("c")
    sid = lax.axis_index("s")
    wid = sid * SC_CORES + cid
    base = wid * BPW

    pltpu.sync_copy(ids_hbm.at[pl.ds(base, GCHUNK)], idx0)
    g0 = pltpu.async_copy(cb_hbm.at[idx0], rows, semg)
    x0 = pltpu.async_copy(x_hbm.at[pl.ds(base, BPW)], xv, semx)
    pltpu.sync_copy(ids_hbm.at[pl.ds(base + GCHUNK, GCHUNK)], idx1)

    def sq_acc(xoff, acc):
        def body(r, a):
            for v in range(D // 16):
                dlt = (xv[xoff + r, pl.ds(16 * v, 16)]
                       - rows[r, pl.ds(16 * v, 16)])
                a = a + dlt * dlt
            return a
        return lax.fori_loop(0, GCHUNK, body, acc)

    g0.wait()
    x0.wait()
    acc = sq_acc(0, jnp.zeros((16,), jnp.float32))
    o0 = pltpu.async_copy(rows, out_hbm.at[pl.ds(base, GCHUNK)], semo)
    o0.wait()
    g1 = pltpu.async_copy(cb_hbm.at[idx1], rows, semg)
    g1.wait()
    acc = sq_acc(GCHUNK, acc)
    o1 = pltpu.async_copy(rows, out_hbm.at[pl.ds(base + GCHUNK, GCHUNK)],
                          semo)
    accv[...] = acc * ((1.0 + COMMIT_W) / B_TOK)
    o1.wait()
    pltpu.sync_copy(accv, part_hbm.at[wid])


def _sc_gather_loss(codebook, x, ids):
    mesh = plsc.VectorSubcoreMesh(
        core_axis_name="c", subcore_axis_name="s",
        num_cores=SC_CORES, num_subcores=SC_SUBCORES,
    )
    return pl.kernel(
        _gather_loss_body,
        out_type=(
            jax.ShapeDtypeStruct((B_TOK, D), jnp.float32),
            jax.ShapeDtypeStruct((NW, 16), jnp.float32),
        ),
        mesh=mesh,
        scratch_types=[
            pltpu.VMEM((GCHUNK,), jnp.int32),
            pltpu.VMEM((GCHUNK,), jnp.int32),
            pltpu.VMEM((GCHUNK, D), jnp.float32),
            pltpu.VMEM((BPW, D), jnp.float32),
            pltpu.VMEM((16,), jnp.float32),
            pltpu.VMEM((16,), jnp.float32),
            pltpu.VMEM_SHARED((SC_SUBCORES, 16), jnp.float32),
            pltpu.SemaphoreType.DMA,
            pltpu.SemaphoreType.DMA,
            pltpu.SemaphoreType.DMA,
        ],
    )(codebook, x, ids)


def kernel(x, temperature, codebook):
    ids = _dist_argmin(x, codebook)
    emb_out, part = _sc_gather_loss(codebook, x, ids)
    loss = jnp.sum(part)
    return emb_out, ids, loss


# speedup vs baseline: 1.1763x; 1.0008x over previous
import jax
import jax.numpy as jnp
from jax import lax
from jax.experimental import pallas as pl
from jax.experimental.pallas import tpu as pltpu
from jax.experimental.pallas import tpu_sc as plsc

B_TOK = 8192
D = 256
K = 8192
COMMIT_W = 0.25

BX = 1024

SC_CORES = 2
SC_SUBCORES = 16
NW = SC_CORES * SC_SUBCORES
BPW = B_TOK // NW
GCHUNK = 128


def _dist_argmin_body(x_ref, cb_ref, ids_ref, x2, cn_sc):
    i = pl.program_id(0)

    xx = x_ref[...]
    x2[...] = xx * -2.0

    c = cb_ref[...]

    @pl.when(i == 0)
    def _():
        cn_sc[...] = jnp.sum(c * c, axis=1, keepdims=True)

    xc2 = lax.dot_general(
        c, x2[...], (((1,), (1,)), ((), ())),
        preferred_element_type=jnp.float32,
        precision=lax.Precision.DEFAULT,
    )
    distT = xc2 + cn_sc[...]
    ids_ref[...] = jnp.argmin(distT, axis=0).astype(jnp.int32)


def _dist_argmin(x, codebook):
    nxb = B_TOK // BX
    return pl.pallas_call(
        _dist_argmin_body,
        grid=(nxb,),
        in_specs=[
            pl.BlockSpec((BX, D), lambda i: (i, 0)),
            pl.BlockSpec((K, D), lambda i: (0, 0)),
        ],
        out_specs=pl.BlockSpec((BX,), lambda i: (i,)),
        out_shape=jax.ShapeDtypeStruct((B_TOK,), jnp.int32),
        scratch_shapes=[
            pltpu.VMEM((BX, D), jnp.float32),
            pltpu.VMEM((K, 1), jnp.float32),
        ],
    )(x, codebook)


def _gather_loss_body(cb_hbm, x_hbm, ids_hbm, out_hbm, part_hbm,
                      idx0, idx1, rows, xv, accv, semg, semx, semo):
    cid = lax.axis_index("c")
    sid = lax.axis_index("s")
    wid = sid * SC_CORES + cid
    base = wid * BPW

    pltpu.sync_copy(ids_hbm.at[pl.ds(base, GCHUNK)], idx0)
    g0 = pltpu.async_copy(cb_hbm.at[idx0], rows, semg)
    x0 = pltpu.async_copy(x_hbm.at[pl.ds(base, BPW)], xv, semx)
    pltpu.sync_copy(ids_hbm.at[pl.ds(base + GCHUNK, GCHUNK)], idx1)

    def sq_acc(xoff, acc):
        def body(r, a):
            for v in range(D // 16):
                dlt = (xv[xoff + r, pl.ds(16 * v, 16)]
                       - rows[r, pl.ds(16 * v, 16)])
                a = a + dlt * dlt
            return a
        return lax.fori_loop(0, GCHUNK, body, acc)

    g0.wait()
    x0.wait()
    acc = sq_acc(0, jnp.zeros((16,), jnp.float32))
    o0 = pltpu.async_copy(rows, out_hbm.at[pl.ds(base, GCHUNK)], semo)
    o0.wait()
    g1 = pltpu.async_copy(cb_hbm.at[idx1], rows, semg)
    g1.wait()
    acc = sq_acc(GCHUNK, acc)
    o1 = pltpu.async_copy(rows, out_hbm.at[pl.ds(base + GCHUNK, GCHUNK)],
                          semo)
    accv[...] = acc * ((1.0 + COMMIT_W) / B_TOK)
    o1.wait()
    pltpu.sync_copy(accv, part_hbm.at[wid])


def _sc_gather_loss(codebook, x, ids):
    mesh = plsc.VectorSubcoreMesh(
        core_axis_name="c", subcore_axis_name="s",
        num_cores=SC_CORES, num_subcores=SC_SUBCORES,
    )
    return pl.kernel(
        _gather_loss_body,
        out_type=(
            jax.ShapeDtypeStruct((B_TOK, D), jnp.float32),
            jax.ShapeDtypeStruct((NW, 16), jnp.float32),
        ),
        mesh=mesh,
        scratch_types=[
            pltpu.VMEM((GCHUNK,), jnp.int32),
            pltpu.VMEM((GCHUNK,), jnp.int32),
            pltpu.VMEM((GCHUNK, D), jnp.float32),
            pltpu.VMEM((BPW, D), jnp.float32),
            pltpu.VMEM((16,), jnp.float32),
            pltpu.SemaphoreType.DMA,
            pltpu.SemaphoreType.DMA,
            pltpu.SemaphoreType.DMA,
        ],
    )(codebook, x, ids)


def kernel(x, temperature, codebook):
    ids = _dist_argmin(x, codebook)
    emb_out, part = _sc_gather_loss(codebook, x, ids)
    loss = jnp.sum(part)
    return emb_out, ids, loss


# speedup vs baseline: 1.1929x; 1.0141x over previous
import jax
import jax.numpy as jnp
from jax import lax
from jax.experimental import pallas as pl
from jax.experimental.pallas import tpu as pltpu
from jax.experimental.pallas import tpu_sc as plsc

B_TOK = 8192
D = 256
K = 8192
COMMIT_W = 0.25

BX = 1024

SC_CORES = 2
SC_SUBCORES = 16
NW = SC_CORES * SC_SUBCORES
BPW = B_TOK // NW
GCHUNK = 128


def _dist_argmin_body(x_ref, cb_ref, ids_ref, x2, cn_sc):
    i = pl.program_id(0)

    xx = x_ref[...]
    x2[...] = xx * -2.0

    c = cb_ref[...]

    @pl.when(i == 0)
    def _():
        cn_sc[...] = jnp.sum(c * c, axis=1, keepdims=True)

    xc2 = lax.dot_general(
        c, x2[...], (((1,), (1,)), ((), ())),
        preferred_element_type=jnp.float32,
        precision=lax.Precision.DEFAULT,
    )
    distT = xc2 + cn_sc[...]
    ids_ref[...] = jnp.argmin(distT, axis=0).astype(jnp.int32)


def _dist_argmin(x, codebook):
    nxb = B_TOK // BX
    return pl.pallas_call(
        _dist_argmin_body,
        grid=(nxb,),
        in_specs=[
            pl.BlockSpec((BX, D), lambda i: (i, 0)),
            pl.BlockSpec((K, D), lambda i: (0, 0)),
        ],
        out_specs=pl.BlockSpec((BX,), lambda i: (i,)),
        out_shape=jax.ShapeDtypeStruct((B_TOK,), jnp.int32),
        scratch_shapes=[
            pltpu.VMEM((BX, D), jnp.float32),
            pltpu.VMEM((K, 1), jnp.float32),
        ],
    )(x, codebook)


def _gather_loss_body(cb_hbm, x_hbm, ids_hbm, out_hbm, part_hbm,
                      idx0, idx1, rows0, rows1, xv, accv,
                      semg0, semg1, semx, semo):
    cid = lax.axis_index("c")
    sid = lax.axis_index("s")
    wid = sid * SC_CORES + cid
    base = wid * BPW

    pltpu.sync_copy(ids_hbm.at[pl.ds(base, GCHUNK)], idx0)
    g0 = pltpu.async_copy(cb_hbm.at[idx0], rows0, semg0)
    pltpu.sync_copy(ids_hbm.at[pl.ds(base + GCHUNK, GCHUNK)], idx1)
    g1 = pltpu.async_copy(cb_hbm.at[idx1], rows1, semg1)
    x0 = pltpu.async_copy(x_hbm.at[pl.ds(base, GCHUNK)], xv, semx)

    def sq_acc(rows, acc):
        def body(r, a):
            for v in range(D // 16):
                dlt = (xv[r, pl.ds(16 * v, 16)]
                       - rows[r, pl.ds(16 * v, 16)])
                a = a + dlt * dlt
            return a
        return lax.fori_loop(0, GCHUNK, body, acc)

    g0.wait()
    x0.wait()
    o0 = pltpu.async_copy(rows0, out_hbm.at[pl.ds(base, GCHUNK)], semo)
    acc = sq_acc(rows0, jnp.zeros((16,), jnp.float32))
    pltpu.sync_copy(x_hbm.at[pl.ds(base + GCHUNK, GCHUNK)], xv)
    g1.wait()
    o1 = pltpu.async_copy(rows1, out_hbm.at[pl.ds(base + GCHUNK, GCHUNK)],
                          semo)
    acc = sq_acc(rows1, acc)
    accv[...] = acc * ((1.0 + COMMIT_W) / B_TOK)
    o0.wait()
    o1.wait()
    pltpu.sync_copy(accv, part_hbm.at[wid])


def _sc_gather_loss(codebook, x, ids):
    mesh = plsc.VectorSubcoreMesh(
        core_axis_name="c", subcore_axis_name="s",
        num_cores=SC_CORES, num_subcores=SC_SUBCORES,
    )
    return pl.kernel(
        _gather_loss_body,
        out_type=(
            jax.ShapeDtypeStruct((B_TOK, D), jnp.float32),
            jax.ShapeDtypeStruct((NW, 16), jnp.float32),
        ),
        mesh=mesh,
        scratch_types=[
            pltpu.VMEM((GCHUNK,), jnp.int32),
            pltpu.VMEM((GCHUNK,), jnp.int32),
            pltpu.VMEM((GCHUNK, D), jnp.float32),
            pltpu.VMEM((GCHUNK, D), jnp.float32),
            pltpu.VMEM((GCHUNK, D), jnp.float32),
            pltpu.VMEM((16,), jnp.float32),
            pltpu.SemaphoreType.DMA,
            pltpu.SemaphoreType.DMA,
            pltpu.SemaphoreType.DMA,
            pltpu.SemaphoreType.DMA,
        ],
    )(codebook, x, ids)


def kernel(x, temperature, codebook):
    ids = _dist_argmin(x, codebook)
    emb_out, part = _sc_gather_loss(codebook, x, ids)
    loss = jnp.sum(part)
    return emb_out, ids, loss
